# Initial kernel scaffold; baseline (speedup 1.0000x reference)
import jax
import jax.numpy as jnp
from jax.experimental import pallas as pl


def kernel(u_pos, v_pos, v_neg, U, V):
    raise NotImplementedError("write your pallas kernel here")


# speedup vs baseline: 5.1417x; 5.1417x over previous
import functools

import jax
import jax.numpy as jnp
from jax import lax
from jax.experimental import pallas as pl
from jax.experimental.pallas import tpu as pltpu
from jax.experimental.pallas import tpu_sc as plsc

VOCAB = 1000000
DIM = 64
B = 16384
NEG = 20

NC = 2
NS = 16
L = 16
NW = NC * NS
BPW = B // NW
C = 32
NCHUNK = BPW // C
NIDX_ROWS = C * NEG // 128


def _sc_partials(u_pos2d, v_pos2d, v_neg2d, U, V):
    mesh = plsc.VectorSubcoreMesh(core_axis_name="c", subcore_axis_name="s")

    @functools.partial(
        pl.kernel,
        mesh=mesh,
        compiler_params=pltpu.CompilerParams(use_tc_tiling_on_sc=False),
        out_type=[
            jax.ShapeDtypeStruct((B, L), jnp.float32),
            jax.ShapeDtypeStruct((B, L), jnp.float32),
        ],
        scratch_types=[
            pltpu.VMEM((NCHUNK, C), jnp.int32),
            pltpu.VMEM((NCHUNK, C), jnp.int32),
            pltpu.VMEM((NCHUNK * NIDX_ROWS, 128), jnp.int32),
            pltpu.VMEM((C, DIM), jnp.float32),
            pltpu.VMEM((C, DIM), jnp.float32),
            pltpu.VMEM((C * NEG, DIM), jnp.float32),
            pltpu.VMEM((BPW, L), jnp.float32),
            pltpu.VMEM((BPW, L), jnp.float32),
            pltpu.SemaphoreType.DMA,
        ],
    )
    def k(up_hbm, vp_hbm, vn_hbm, u_hbm, v_hbm, p_hbm, q_hbm,
          uidx, vidx, nidx, eu, ev, nrows, pw, qw, sem):
        wid = lax.axis_index("s") * NC + lax.axis_index("c")
        base = wid * BPW
        pltpu.sync_copy(up_hbm.at[pl.ds(wid * NCHUNK, NCHUNK)], uidx)
        pltpu.sync_copy(vp_hbm.at[pl.ds(wid * NCHUNK, NCHUNK)], vidx)
        pltpu.sync_copy(
            vn_hbm.at[pl.ds(wid * NCHUNK * NIDX_ROWS, NCHUNK * NIDX_ROWS)],
            nidx)

        def chunk_body(c, _):
            copies = [
                pltpu.async_copy(u_hbm.at[uidx.at[c]], eu, sem),
                pltpu.async_copy(v_hbm.at[vidx.at[c]], ev, sem),
            ]
            for j in range(NIDX_ROWS):
                copies.append(pltpu.async_copy(
                    v_hbm.at[nidx.at[c * NIDX_ROWS + j]],
                    nrows.at[pl.ds(j * 128, 128)], sem))
            for cp in copies:
                cp.wait()

            def elem_body(i, _):
                e = [eu[i, pl.ds(kk * L, L)] for kk in range(4)]
                p = e[0] * ev[i, pl.ds(0, L)]
                for kk in range(1, 4):
                    p = p + e[kk] * ev[i, pl.ds(kk * L, L)]
                q = p - p
                for n in range(NEG):
                    r = i * NEG + n
                    for kk in range(4):
                        q = q + e[kk] * nrows[r, pl.ds(kk * L, L)]
                pw[c * C + i, pl.ds(0, L)] = p
                qw[c * C + i, pl.ds(0, L)] = q
                return 0

            lax.fori_loop(0, C, elem_body, 0, unroll=False)
            return 0

        lax.fori_loop(0, NCHUNK, chunk_body, 0, unroll=False)
        pltpu.sync_copy(pw, p_hbm.at[pl.ds(base, BPW)])
        pltpu.sync_copy(qw, q_hbm.at[pl.ds(base, BPW)])

    return k(u_pos2d, v_pos2d, v_neg2d, U, V)


def _tc_loss_body(p_ref, q_ref, o_ref):
    a = jnp.sum(p_ref[...], axis=1, keepdims=True)
    s = jnp.sum(q_ref[...], axis=1, keepdims=True)

    def log_sigmoid(x):
        return jnp.minimum(x, 0.0) - jnp.log1p(jnp.exp(-jnp.abs(x)))

    ls = log_sigmoid(a) + log_sigmoid(-s)
    o_ref[...] = jnp.reshape(-jnp.sum(ls) / B, (1, 1))


def kernel(u_pos, v_pos, v_neg, U, V):
    u_pos2d = u_pos.astype(jnp.int32).reshape(B // C, C)
    v_pos2d = v_pos.astype(jnp.int32).reshape(B // C, C)
    v_neg2d = v_neg.astype(jnp.int32).reshape(B * NEG // 128, 128)
    P, Q = _sc_partials(u_pos2d, v_pos2d, v_neg2d, U, V)
    loss = pl.pallas_call(
        _tc_loss_body,
        out_shape=jax.ShapeDtypeStruct((1, 1), jnp.float32),
    )(P, Q)
    return loss[0, 0]
